# Initial kernel scaffold; baseline (speedup 1.0000x reference)
import functools

import jax
import jax.numpy as jnp
from jax import lax
from jax.experimental import pallas as pl
from jax.experimental.pallas import tpu as pltpu
import jax.experimental.pallas.tpu_sc as plsc

N = 10000
E = 320000
D = 128
NGRP = 25
GRP = 400

NC = 2
NS = 16
NW = NC * NS
EPW = E // NW
CHUNK = 80
NCHUNK = EPW // CHUNK

DEG_PAD = 10240
M_PER_TILE = 15632
M_PAD = M_PER_TILE * NS
ROWS_PER_TILE = N // NS

_HIGH = jax.lax.Precision.HIGHEST

_mesh = plsc.VectorSubcoreMesh(core_axis_name="c", subcore_axis_name="s")


@functools.partial(
    pl.kernel,
    out_type=jax.ShapeDtypeStruct((NC * DEG_PAD,), jnp.float32),
    mesh=_mesh,
    scratch_types=[
        pltpu.VMEM((CHUNK,), jnp.int32),
        pltpu.VMEM((CHUNK,), jnp.float32),
        pltpu.VMEM((640,), jnp.float32),
        pltpu.VMEM_SHARED((DEG_PAD,), jnp.float32),
    ],
)
def _deg_kernel(dst_hbm, out_hbm, idx_v, ones_v, zero_v, acc_sh):
    c = lax.axis_index("c")
    s = lax.axis_index("s")
    wid = s * NC + c

    for j in range(CHUNK // 16):
        ones_v[pl.ds(j * 16, 16)] = jnp.full((16,), 1.0, jnp.float32)
    for j in range(640 // 16):
        zero_v[pl.ds(j * 16, 16)] = jnp.zeros((16,), jnp.float32)
    pltpu.sync_copy(zero_v, acc_sh.at[pl.ds(s * 640, 640)])
    plsc.subcore_barrier()

    def body(k, carry):
        base = wid * EPW + k * CHUNK
        pltpu.sync_copy(dst_hbm.at[pl.ds(base, CHUNK)], idx_v)
        pltpu.sync_copy(ones_v, acc_sh.at[idx_v], add=True)
        return carry

    lax.fori_loop(0, NCHUNK, body, 0)
    plsc.subcore_barrier()
    pltpu.sync_copy(
        acc_sh.at[pl.ds(s * 640, 640)],
        out_hbm.at[pl.ds(c * DEG_PAD + s * 640, 640)],
    )


@functools.partial(
    pl.kernel,
    out_type=(
        jax.ShapeDtypeStruct((NC * N, D), jnp.float32),
        jax.ShapeDtypeStruct((NC * M_PAD,), jnp.float32),
    ),
    mesh=_mesh,
    scratch_types=[
        pltpu.VMEM((CHUNK,), jnp.int32),
        pltpu.VMEM((CHUNK,), jnp.int32),
        pltpu.VMEM((CHUNK, D), jnp.float32),
        pltpu.VMEM((DEG_PAD,), jnp.float32),
        pltpu.VMEM((CHUNK,), jnp.float32),
        pltpu.VMEM((CHUNK,), jnp.int32),
        pltpu.VMEM_SHARED((N, D), jnp.float32),
        pltpu.VMEM_SHARED((M_PAD,), jnp.float32),
        pltpu.SemaphoreType.DMA,
    ],
)
def _edge_kernel(src_hbm, dst_hbm, g1_hbm, dinv_hbm, z2d_hbm, z1d_hbm,
                 s1_hbm, m_hbm,
                 idx_s, idx_d, rows_v, dinv_v, mval_v, midx_v,
                 acc_sh, m_sh, sem):
    c = lax.axis_index("c")
    s = lax.axis_index("s")
    wid = s * NC + c

    pltpu.sync_copy(dinv_hbm, dinv_v)
    pltpu.sync_copy(
        z2d_hbm.at[pl.ds(s * ROWS_PER_TILE, ROWS_PER_TILE)],
        acc_sh.at[pl.ds(s * ROWS_PER_TILE, ROWS_PER_TILE)],
    )
    pltpu.sync_copy(
        z1d_hbm.at[pl.ds(s * M_PER_TILE, M_PER_TILE)],
        m_sh.at[pl.ds(s * M_PER_TILE, M_PER_TILE)],
    )
    plsc.subcore_barrier()

    def body(k, carry):
        base = wid * EPW + k * CHUNK
        pltpu.sync_copy(src_hbm.at[pl.ds(base, CHUNK)], idx_s)
        pltpu.sync_copy(dst_hbm.at[pl.ds(base, CHUNK)], idx_d)
        pltpu.async_copy(g1_hbm.at[idx_s], rows_v, sem).wait()
        pltpu.sync_copy(rows_v, acc_sh.at[idx_d], add=True)
        for j in range(CHUNK // 16):
            sl = pl.ds(j * 16, 16)
            d16 = idx_d[sl]
            s16 = idx_s[sl]
            dv = plsc.load_gather(dinv_v, [d16])
            grp = d16 // GRP
            mval_v[sl] = dv
            midx_v[sl] = grp * N + s16
        pltpu.sync_copy(mval_v, m_sh.at[midx_v], add=True)
        return carry

    lax.fori_loop(0, NCHUNK, body, 0)
    plsc.subcore_barrier()
    pltpu.sync_copy(
        acc_sh.at[pl.ds(s * ROWS_PER_TILE, ROWS_PER_TILE)],
        s1_hbm.at[pl.ds(c * N + s * ROWS_PER_TILE, ROWS_PER_TILE)],
    )
    pltpu.sync_copy(
        m_sh.at[pl.ds(s * M_PER_TILE, M_PER_TILE)],
        m_hbm.at[pl.ds(c * M_PAD + s * M_PER_TILE, M_PER_TILE)],
    )


def _tc_pre_body(x_ref, w1_ref, dp0_ref, dp1_ref,
                 g1_ref, hself_ref, dinv_ref, invdeg_ref):
    h1 = jnp.dot(x_ref[...], w1_ref[...],
                 preferred_element_type=jnp.float32, precision=_HIGH)
    deg = dp0_ref[...] + dp1_ref[...] + 1.0
    dinv = 1.0 / jnp.sqrt(deg)
    invdeg = 1.0 / deg
    dinv_ref[...] = dinv
    invdeg_ref[...] = invdeg
    g1_ref[...] = h1 * dinv[:N]
    hself_ref[...] = h1 * invdeg[:N]


_tc_pre = pl.pallas_call(
    _tc_pre_body,
    out_shape=(
        jax.ShapeDtypeStruct((N, D), jnp.float32),
        jax.ShapeDtypeStruct((N, D), jnp.float32),
        jax.ShapeDtypeStruct((DEG_PAD, 1), jnp.float32),
        jax.ShapeDtypeStruct((DEG_PAD, 1), jnp.float32),
    ),
)


def _tc_post_body(s0_ref, s1_ref, hself_ref, dinv_ref, invdeg_ref,
                  m0_ref, m1_ref, w2_ref, b1_ref, b2_ref, out_ref):
    dinv = dinv_ref[...][:N]
    invdeg = invdeg_ref[...][:N]
    s1 = s0_ref[...] + s1_ref[...]
    hr = jnp.maximum(dinv * s1 + hself_ref[...] + b1_ref[...], 0.0)
    t = dinv * hr
    u = invdeg * hr
    m = m0_ref[...] + m1_ref[...]
    gi = lax.broadcasted_iota(jnp.int32, (NGRP, N), 0)
    vi = lax.broadcasted_iota(jnp.int32, (NGRP, N), 1)
    diff = vi - gi * GRP
    p = jnp.where((diff >= 0) & (diff < GRP), 1.0, 0.0).astype(jnp.float32)
    agg = (jnp.dot(m, t, preferred_element_type=jnp.float32, precision=_HIGH)
           + jnp.dot(p, u, preferred_element_type=jnp.float32,
                     precision=_HIGH))
    out = jnp.dot(agg, w2_ref[...], preferred_element_type=jnp.float32,
                  precision=_HIGH)
    out_ref[...] = out * (1.0 / GRP) + b2_ref[...]


_tc_post = pl.pallas_call(
    _tc_post_body,
    out_shape=jax.ShapeDtypeStruct((NGRP, 40), jnp.float32),
)


def kernel(x, edge_index, W1, b1, W2, b2):
    src = edge_index[0]
    dst = edge_index[1]

    degp = _deg_kernel(dst)
    dp = degp.reshape(NC, DEG_PAD, 1)
    g1, hself, dinv2d, invdeg2d = _tc_pre(x, W1, dp[0], dp[1])

    z2d = jnp.zeros((N, D), jnp.float32)
    z1d = jnp.zeros((M_PAD,), jnp.float32)
    s1p, mp = _edge_kernel(src, dst, g1, dinv2d.reshape(DEG_PAD), z2d, z1d)

    s1p = s1p.reshape(NC, N, D)
    mp = mp.reshape(NC, M_PAD)
    m0 = mp[0, : NGRP * N].reshape(NGRP, N)
    m1 = mp[1, : NGRP * N].reshape(NGRP, N)

    out = _tc_post(s1p[0], s1p[1], hself, dinv2d, invdeg2d,
                   m0, m1, W2, b1.reshape(1, D), b2.reshape(1, 40))
    return out


# speedup vs baseline: 18.6529x; 18.6529x over previous
import functools

import jax
import jax.numpy as jnp
from jax import lax
from jax.experimental import pallas as pl
from jax.experimental.pallas import tpu as pltpu
import jax.experimental.pallas.tpu_sc as plsc

N = 10000
E = 320000
D = 128
NGRP = 25
GRP = 400

NC = 2
NS = 16
NW = NC * NS
EPW = E // NW
CHUNK = 80
NCHUNK = EPW // CHUNK

DEG_PAD = 10240
M_PER_TILE = 16000
M_CHUNK = 640
M_PAD = M_PER_TILE * NS
ROWS_PER_TILE = 632
ROWS_PAD = ROWS_PER_TILE * NS

_HIGH = jax.lax.Precision.HIGHEST

_mesh = plsc.VectorSubcoreMesh(core_axis_name="c", subcore_axis_name="s")


@functools.partial(
    pl.kernel,
    out_type=jax.ShapeDtypeStruct((NC * DEG_PAD,), jnp.float32),
    mesh=_mesh,
    scratch_types=[
        pltpu.VMEM((CHUNK,), jnp.int32),
        pltpu.VMEM((CHUNK,), jnp.float32),
        pltpu.VMEM((640,), jnp.float32),
        pltpu.VMEM_SHARED((DEG_PAD,), jnp.float32),
    ],
)
def _deg_kernel(dst_hbm, out_hbm, idx_v, ones_v, zero_v, acc_sh):
    c = lax.axis_index("c")
    s = lax.axis_index("s")
    wid = s * NC + c

    for j in range(CHUNK // 16):
        ones_v[pl.ds(j * 16, 16)] = jnp.full((16,), 1.0, jnp.float32)
    for j in range(640 // 16):
        zero_v[pl.ds(j * 16, 16)] = jnp.zeros((16,), jnp.float32)
    pltpu.sync_copy(zero_v, acc_sh.at[pl.ds(s * 640, 640)])
    plsc.subcore_barrier()

    def body(k, carry):
        base = wid * EPW + k * CHUNK
        pltpu.sync_copy(dst_hbm.at[pl.ds(base, CHUNK)], idx_v)
        pltpu.sync_copy(ones_v, acc_sh.at[idx_v], add=True)
        return carry

    lax.fori_loop(0, NCHUNK, body, 0)
    plsc.subcore_barrier()
    pltpu.sync_copy(
        acc_sh.at[pl.ds(s * 640, 640)],
        out_hbm.at[pl.ds(c * DEG_PAD + s * 640, 640)],
    )


@functools.partial(
    pl.kernel,
    out_type=(
        jax.ShapeDtypeStruct((NC * ROWS_PAD, D), jnp.float32),
        jax.ShapeDtypeStruct((NC * M_PAD,), jnp.float32),
    ),
    mesh=_mesh,
    scratch_types=[
        pltpu.VMEM((CHUNK,), jnp.int32),
        pltpu.VMEM((CHUNK,), jnp.int32),
        pltpu.VMEM((CHUNK, D), jnp.float32),
        pltpu.VMEM((CHUNK,), jnp.float32),
        pltpu.VMEM((CHUNK,), jnp.int32),
        pltpu.VMEM_SHARED((ROWS_PAD, D), jnp.float32),
        pltpu.VMEM_SHARED((M_PAD,), jnp.float32),
        pltpu.SemaphoreType.DMA,
        pltpu.SemaphoreType.DMA,
    ],
)
def _edge_kernel(src_hbm, dst_hbm, g1_hbm, dinv_hbm, z2d_hbm, z1d_hbm,
                 s1_hbm, m_hbm,
                 idx_s, idx_d, rows_v, mval_v, midx_v,
                 acc_sh, m_sh, sem, sem2):
    c = lax.axis_index("c")
    s = lax.axis_index("s")
    wid = s * NC + c

    pltpu.sync_copy(
        z2d_hbm.at[pl.ds(s * ROWS_PER_TILE, ROWS_PER_TILE)],
        acc_sh.at[pl.ds(s * ROWS_PER_TILE, ROWS_PER_TILE)],
    )
    for t in range(M_PER_TILE // M_CHUNK):
        pltpu.sync_copy(
            z1d_hbm.at[pl.ds(s * M_PER_TILE + t * M_CHUNK, M_CHUNK)],
            m_sh.at[pl.ds(s * M_PER_TILE + t * M_CHUNK, M_CHUNK)],
        )
    plsc.subcore_barrier()

    def body(k, carry):
        base = wid * EPW + k * CHUNK
        pltpu.sync_copy(src_hbm.at[pl.ds(base, CHUNK)], idx_s)
        pltpu.sync_copy(dst_hbm.at[pl.ds(base, CHUNK)], idx_d)
        gcp = pltpu.async_copy(g1_hbm.at[idx_s], rows_v, sem)
        dcp = pltpu.async_copy(dinv_hbm.at[idx_d], mval_v, sem2)
        gcp.wait()
        pltpu.sync_copy(rows_v, acc_sh.at[idx_d], add=True)
        for j in range(CHUNK // 16):
            sl = pl.ds(j * 16, 16)
            d16 = idx_d[sl]
            s16 = idx_s[sl]
            grp = lax.shift_right_logical(
                d16 * 10486, jnp.full((16,), 22, jnp.int32))
            midx_v[sl] = grp * N + s16
        dcp.wait()
        pltpu.sync_copy(mval_v, m_sh.at[midx_v], add=True)
        return carry

    lax.fori_loop(0, NCHUNK, body, 0)
    plsc.subcore_barrier()
    pltpu.sync_copy(
        acc_sh.at[pl.ds(s * ROWS_PER_TILE, ROWS_PER_TILE)],
        s1_hbm.at[pl.ds(c * ROWS_PAD + s * ROWS_PER_TILE, ROWS_PER_TILE)],
    )
    for t in range(M_PER_TILE // M_CHUNK):
        pltpu.sync_copy(
            m_sh.at[pl.ds(s * M_PER_TILE + t * M_CHUNK, M_CHUNK)],
            m_hbm.at[pl.ds(c * M_PAD + s * M_PER_TILE + t * M_CHUNK, M_CHUNK)],
        )


_PRE_BLK = 2000
_PRE_GRID = N // _PRE_BLK


def _tc_pre_body(x_ref, w1_ref, dp0b_ref, dp1b_ref, dp0f_ref, dp1f_ref,
                 g1_ref, hself_ref, dinv_ref, invdeg_ref):
    i = pl.program_id(0)
    h1 = jnp.dot(x_ref[...], w1_ref[...],
                 preferred_element_type=jnp.float32, precision=_HIGH)
    degb = dp0b_ref[...] + dp1b_ref[...] + 1.0
    g1_ref[...] = h1 * (1.0 / jnp.sqrt(degb))
    hself_ref[...] = h1 * (1.0 / degb)

    @pl.when(i == 0)
    def _():
        degf = dp0f_ref[...] + dp1f_ref[...] + 1.0
        dinv_ref[...] = 1.0 / jnp.sqrt(degf)
        invdeg_ref[...] = 1.0 / degf


_tc_pre = pl.pallas_call(
    _tc_pre_body,
    grid=(_PRE_GRID,),
    in_specs=[
        pl.BlockSpec((_PRE_BLK, D), lambda i: (i, 0)),
        pl.BlockSpec((D, D), lambda i: (0, 0)),
        pl.BlockSpec((_PRE_BLK, 1), lambda i: (i, 0)),
        pl.BlockSpec((_PRE_BLK, 1), lambda i: (i, 0)),
        pl.BlockSpec((DEG_PAD, 1), lambda i: (0, 0)),
        pl.BlockSpec((DEG_PAD, 1), lambda i: (0, 0)),
    ],
    out_specs=(
        pl.BlockSpec((_PRE_BLK, D), lambda i: (i, 0)),
        pl.BlockSpec((_PRE_BLK, D), lambda i: (i, 0)),
        pl.BlockSpec((DEG_PAD, 1), lambda i: (0, 0)),
        pl.BlockSpec((DEG_PAD, 1), lambda i: (0, 0)),
    ),
    out_shape=(
        jax.ShapeDtypeStruct((N, D), jnp.float32),
        jax.ShapeDtypeStruct((N, D), jnp.float32),
        jax.ShapeDtypeStruct((DEG_PAD, 1), jnp.float32),
        jax.ShapeDtypeStruct((DEG_PAD, 1), jnp.float32),
    ),
)


def _tc_post_body(s0_ref, s1_ref, hself_ref, dinv_ref, invdeg_ref,
                  m0_ref, m1_ref, w2_ref, b1_ref, b2_ref, out_ref):
    dinv = dinv_ref[...][:N]
    invdeg = invdeg_ref[...][:N]
    s1 = s0_ref[...] + s1_ref[...]
    hr = jnp.maximum(dinv * s1 + hself_ref[...] + b1_ref[...], 0.0)
    t = dinv * hr
    u = invdeg * hr
    m = m0_ref[...] + m1_ref[...]
    gi = lax.broadcasted_iota(jnp.int32, (NGRP, N), 0)
    vi = lax.broadcasted_iota(jnp.int32, (NGRP, N), 1)
    diff = vi - gi * GRP
    p = jnp.where((diff >= 0) & (diff < GRP), 1.0, 0.0).astype(jnp.float32)
    agg = (jnp.dot(m, t, preferred_element_type=jnp.float32, precision=_HIGH)
           + jnp.dot(p, u, preferred_element_type=jnp.float32,
                     precision=_HIGH))
    out = jnp.dot(agg, w2_ref[...], preferred_element_type=jnp.float32,
                  precision=_HIGH)
    out_ref[...] = out * (1.0 / GRP) + b2_ref[...]


_tc_post = pl.pallas_call(
    _tc_post_body,
    out_shape=jax.ShapeDtypeStruct((NGRP, 40), jnp.float32),
)


def kernel(x, edge_index, W1, b1, W2, b2):
    src = edge_index[0]
    dst = edge_index[1]

    degp = _deg_kernel(dst)
    dp = degp.reshape(NC, DEG_PAD, 1)
    g1, hself, dinv2d, invdeg2d = _tc_pre(x, W1, dp[0], dp[1], dp[0], dp[1])

    z2d = jnp.zeros((ROWS_PAD, D), jnp.float32)
    z1d = jnp.zeros((M_PAD,), jnp.float32)
    s1p, mp = _edge_kernel(src, dst, g1, dinv2d.reshape(DEG_PAD), z2d, z1d)

    s1p = s1p.reshape(NC, ROWS_PAD, D)
    mp = mp.reshape(NC, M_PAD)
    m0 = mp[0, : NGRP * N].reshape(NGRP, N)
    m1 = mp[1, : NGRP * N].reshape(NGRP, N)

    out = _tc_post(s1p[0, :N], s1p[1, :N], hself, dinv2d, invdeg2d,
                   m0, m1, W2, b1.reshape(1, D), b2.reshape(1, 40))
    return out


# speedup vs baseline: 28.6342x; 1.5351x over previous
import functools

import jax
import jax.numpy as jnp
from jax import lax
from jax.experimental import pallas as pl
from jax.experimental.pallas import tpu as pltpu
import jax.experimental.pallas.tpu_sc as plsc

N = 10000
E = 320000
D = 128
NGRP = 25
GRP = 400

NC = 2
NS = 16
NW = NC * NS
EPW = E // NW
CHUNK = 80
NCHUNK = EPW // CHUNK
IDXBLK = 25
NBLK = NCHUNK // IDXBLK

DEG_PAD = 10240
M_PER_TILE = 16000
M_CHUNK = 640
M_PAD = M_PER_TILE * NS
ROWS_PER_TILE = 632
ROWS_PAD = ROWS_PER_TILE * NS

_HIGH = jax.lax.Precision.HIGHEST

_mesh = plsc.VectorSubcoreMesh(core_axis_name="c", subcore_axis_name="s")


@functools.partial(
    pl.kernel,
    out_type=jax.ShapeDtypeStruct((NC * DEG_PAD,), jnp.float32),
    mesh=_mesh,
    scratch_types=[
        pltpu.VMEM((CHUNK,), jnp.int32),
        pltpu.VMEM((CHUNK,), jnp.float32),
        pltpu.VMEM((640,), jnp.float32),
        pltpu.VMEM_SHARED((DEG_PAD,), jnp.float32),
    ],
)
def _deg_kernel(dst_hbm, out_hbm, idx_v, ones_v, zero_v, acc_sh):
    c = lax.axis_index("c")
    s = lax.axis_index("s")
    wid = s * NC + c

    for j in range(CHUNK // 16):
        ones_v[pl.ds(j * 16, 16)] = jnp.full((16,), 1.0, jnp.float32)
    for j in range(640 // 16):
        zero_v[pl.ds(j * 16, 16)] = jnp.zeros((16,), jnp.float32)
    pltpu.sync_copy(zero_v, acc_sh.at[pl.ds(s * 640, 640)])
    plsc.subcore_barrier()

    def body(k, carry):
        base = wid * EPW + k * CHUNK
        pltpu.sync_copy(dst_hbm.at[pl.ds(base, CHUNK)], idx_v)
        pltpu.sync_copy(ones_v, acc_sh.at[idx_v], add=True)
        return carry

    lax.fori_loop(0, NCHUNK, body, 0)
    plsc.subcore_barrier()
    pltpu.sync_copy(
        acc_sh.at[pl.ds(s * 640, 640)],
        out_hbm.at[pl.ds(c * DEG_PAD + s * 640, 640)],
    )


@functools.partial(
    pl.kernel,
    out_type=(
        jax.ShapeDtypeStruct((NC * ROWS_PAD, D), jnp.float32),
        jax.ShapeDtypeStruct((NC * M_PAD,), jnp.float32),
    ),
    mesh=_mesh,
    scratch_types=[
        pltpu.VMEM((IDXBLK, CHUNK), jnp.int32),
        pltpu.VMEM((IDXBLK, CHUNK), jnp.int32),
        pltpu.VMEM((CHUNK, D), jnp.float32),
        pltpu.VMEM((CHUNK, D), jnp.float32),
        pltpu.VMEM((CHUNK,), jnp.float32),
        pltpu.VMEM((CHUNK,), jnp.float32),
        pltpu.VMEM((CHUNK,), jnp.int32),
        pltpu.VMEM_SHARED((ROWS_PAD, D), jnp.float32),
        pltpu.VMEM_SHARED((M_PAD,), jnp.float32),
        pltpu.SemaphoreType.DMA,
        pltpu.SemaphoreType.DMA,
        pltpu.SemaphoreType.DMA,
        pltpu.SemaphoreType.DMA,
    ],
)
def _edge_kernel(src_hbm, dst_hbm, g1_hbm, dinv_hbm, z2d_hbm, z1d_hbm,
                 s1_hbm, m_hbm,
                 idx_s, idx_d, rows0, rows1, dval0, dval1, midx_v,
                 acc_sh, m_sh, sg0, sg1, sv0, sv1):
    c = lax.axis_index("c")
    s = lax.axis_index("s")
    wid = s * NC + c

    pltpu.sync_copy(
        z2d_hbm.at[pl.ds(s * ROWS_PER_TILE, ROWS_PER_TILE)],
        acc_sh.at[pl.ds(s * ROWS_PER_TILE, ROWS_PER_TILE)],
    )
    for t in range(M_PER_TILE // M_CHUNK):
        pltpu.sync_copy(
            z1d_hbm.at[pl.ds(s * M_PER_TILE + t * M_CHUNK, M_CHUNK)],
            m_sh.at[pl.ds(s * M_PER_TILE + t * M_CHUNK, M_CHUNK)],
        )
    plsc.subcore_barrier()

    def _issue(k, rows, dval, sg, sv):
        g = pltpu.async_copy(g1_hbm.at[idx_s.at[k]], rows, sg)
        v = pltpu.async_copy(dinv_hbm.at[idx_d.at[k]], dval, sv)
        return g, v

    def _consume(k, rows, dval, sg, sv):
        pltpu.make_async_copy(g1_hbm.at[idx_s.at[k]], rows, sg).wait()
        pltpu.sync_copy(rows, acc_sh.at[idx_d.at[k]], add=True)
        for j in range(CHUNK // 16):
            sl = pl.ds(j * 16, 16)
            d16 = idx_d[k, sl]
            s16 = idx_s[k, sl]
            grp = lax.shift_right_logical(
                d16 * 10486, jnp.full((16,), 22, jnp.int32))
            midx_v[sl] = grp * N + s16
        pltpu.make_async_copy(dinv_hbm.at[idx_d.at[k]], dval, sv).wait()
        pltpu.sync_copy(dval, m_sh.at[midx_v], add=True)

    def blk(b, carry):
        pltpu.sync_copy(src_hbm.at[wid, b], idx_s)
        pltpu.sync_copy(dst_hbm.at[wid, b], idx_d)
        _issue(0, rows0, dval0, sg0, sv0)

        def body(i, carry):
            a = 2 * i
            _issue(a + 1, rows1, dval1, sg1, sv1)
            _consume(a, rows0, dval0, sg0, sv0)
            _issue(a + 2, rows0, dval0, sg0, sv0)
            _consume(a + 1, rows1, dval1, sg1, sv1)
            return carry

        lax.fori_loop(0, (IDXBLK - 1) // 2, body, 0)
        _consume(IDXBLK - 1, rows0, dval0, sg0, sv0)
        return carry

    lax.fori_loop(0, NBLK, blk, 0)
    plsc.subcore_barrier()
    pltpu.sync_copy(
        acc_sh.at[pl.ds(s * ROWS_PER_TILE, ROWS_PER_TILE)],
        s1_hbm.at[pl.ds(c * ROWS_PAD + s * ROWS_PER_TILE, ROWS_PER_TILE)],
    )
    for t in range(M_PER_TILE // M_CHUNK):
        pltpu.sync_copy(
            m_sh.at[pl.ds(s * M_PER_TILE + t * M_CHUNK, M_CHUNK)],
            m_hbm.at[pl.ds(c * M_PAD + s * M_PER_TILE + t * M_CHUNK, M_CHUNK)],
        )


_PRE_BLK = 2000
_PRE_GRID = N // _PRE_BLK


def _tc_pre_body(x_ref, w1_ref, dp0b_ref, dp1b_ref, dp0f_ref, dp1f_ref,
                 g1_ref, hself_ref, dinv_ref, invdeg_ref):
    i = pl.program_id(0)
    h1 = jnp.dot(x_ref[...], w1_ref[...],
                 preferred_element_type=jnp.float32, precision=_HIGH)
    degb = dp0b_ref[...] + dp1b_ref[...] + 1.0
    g1_ref[...] = h1 * (1.0 / jnp.sqrt(degb))
    hself_ref[...] = h1 * (1.0 / degb)

    @pl.when(i == 0)
    def _():
        degf = dp0f_ref[...] + dp1f_ref[...] + 1.0
        dinv_ref[...] = 1.0 / jnp.sqrt(degf)
        invdeg_ref[...] = 1.0 / degf


_tc_pre = pl.pallas_call(
    _tc_pre_body,
    grid=(_PRE_GRID,),
    in_specs=[
        pl.BlockSpec((_PRE_BLK, D), lambda i: (i, 0)),
        pl.BlockSpec((D, D), lambda i: (0, 0)),
        pl.BlockSpec((_PRE_BLK, 1), lambda i: (i, 0)),
        pl.BlockSpec((_PRE_BLK, 1), lambda i: (i, 0)),
        pl.BlockSpec((DEG_PAD, 1), lambda i: (0, 0)),
        pl.BlockSpec((DEG_PAD, 1), lambda i: (0, 0)),
    ],
    out_specs=(
        pl.BlockSpec((_PRE_BLK, D), lambda i: (i, 0)),
        pl.BlockSpec((_PRE_BLK, D), lambda i: (i, 0)),
        pl.BlockSpec((DEG_PAD, 1), lambda i: (0, 0)),
        pl.BlockSpec((DEG_PAD, 1), lambda i: (0, 0)),
    ),
    out_shape=(
        jax.ShapeDtypeStruct((N, D), jnp.float32),
        jax.ShapeDtypeStruct((N, D), jnp.float32),
        jax.ShapeDtypeStruct((DEG_PAD, 1), jnp.float32),
        jax.ShapeDtypeStruct((DEG_PAD, 1), jnp.float32),
    ),
)


def _tc_post_body(s0_ref, s1_ref, hself_ref, dinv_ref, invdeg_ref,
                  m0_ref, m1_ref, w2_ref, b1_ref, b2_ref, out_ref):
    dinv = dinv_ref[...][:N]
    invdeg = invdeg_ref[...][:N]
    s1 = s0_ref[...] + s1_ref[...]
    hr = jnp.maximum(dinv * s1 + hself_ref[...] + b1_ref[...], 0.0)
    t = dinv * hr
    u = invdeg * hr
    m = m0_ref[...] + m1_ref[...]
    gi = lax.broadcasted_iota(jnp.int32, (NGRP, N), 0)
    vi = lax.broadcasted_iota(jnp.int32, (NGRP, N), 1)
    diff = vi - gi * GRP
    p = jnp.where((diff >= 0) & (diff < GRP), 1.0, 0.0).astype(jnp.float32)
    agg = (jnp.dot(m, t, preferred_element_type=jnp.float32, precision=_HIGH)
           + jnp.dot(p, u, preferred_element_type=jnp.float32,
                     precision=_HIGH))
    out = jnp.dot(agg, w2_ref[...], preferred_element_type=jnp.float32,
                  precision=_HIGH)
    out_ref[...] = out * (1.0 / GRP) + b2_ref[...]


_tc_post = pl.pallas_call(
    _tc_post_body,
    out_shape=jax.ShapeDtypeStruct((NGRP, 40), jnp.float32),
)


def kernel(x, edge_index, W1, b1, W2, b2):
    src = edge_index[0].reshape(NW, NBLK, IDXBLK, CHUNK)
    dst = edge_index[1]
    dst3 = dst.reshape(NW, NBLK, IDXBLK, CHUNK)

    degp = _deg_kernel(dst)
    dp = degp.reshape(NC, DEG_PAD, 1)
    g1, hself, dinv2d, invdeg2d = _tc_pre(x, W1, dp[0], dp[1], dp[0], dp[1])

    z2d = jnp.zeros((ROWS_PAD, D), jnp.float32)
    z1d = jnp.zeros((M_PAD,), jnp.float32)
    s1p, mp = _edge_kernel(src, dst3, g1, dinv2d.reshape(DEG_PAD), z2d, z1d)

    s1p = s1p.reshape(NC, ROWS_PAD, D)
    mp = mp.reshape(NC, M_PAD)
    m0 = mp[0, : NGRP * N].reshape(NGRP, N)
    m1 = mp[1, : NGRP * N].reshape(NGRP, N)

    out = _tc_post(s1p[0, :N], s1p[1, :N], hself, dinv2d, invdeg2d,
                   m0, m1, W2, b1.reshape(1, D), b2.reshape(1, 40))
    return out


# speedup vs baseline: 29.5611x; 1.0324x over previous
import functools

import jax
import jax.numpy as jnp
from jax import lax
from jax.experimental import pallas as pl
from jax.experimental.pallas import tpu as pltpu
import jax.experimental.pallas.tpu_sc as plsc

N = 10000
E = 320000
D = 128
NGRP = 25
GRP = 400

NC = 2
NS = 16
NW = NC * NS
EPW = E // NW
CHUNK = 80
NCHUNK = EPW // CHUNK
IDXBLK = 25
NBLK = NCHUNK // IDXBLK

DEG_PAD = 10240
M_PER_TILE = 16000
M_CHUNK = 640
M_PAD = M_PER_TILE * NS
ROWS_PER_TILE = 632
ROWS_PAD = ROWS_PER_TILE * NS

_HIGH = jax.lax.Precision.HIGHEST

_mesh = plsc.VectorSubcoreMesh(core_axis_name="c", subcore_axis_name="s")


@functools.partial(
    pl.kernel,
    out_type=jax.ShapeDtypeStruct((NC * DEG_PAD,), jnp.float32),
    mesh=_mesh,
    scratch_types=[
        pltpu.VMEM((IDXBLK, CHUNK), jnp.int32),
        pltpu.VMEM((CHUNK,), jnp.float32),
        pltpu.VMEM((640,), jnp.float32),
        pltpu.VMEM_SHARED((DEG_PAD,), jnp.float32),
        pltpu.SemaphoreType.DMA,
    ],
)
def _deg_kernel(dst_hbm, out_hbm, idx_v, ones_v, zero_v, acc_sh, sem_s):
    c = lax.axis_index("c")
    s = lax.axis_index("s")
    wid = s * NC + c

    for j in range(CHUNK // 16):
        ones_v[pl.ds(j * 16, 16)] = jnp.full((16,), 1.0, jnp.float32)
    for j in range(640 // 16):
        zero_v[pl.ds(j * 16, 16)] = jnp.zeros((16,), jnp.float32)
    pltpu.sync_copy(zero_v, acc_sh.at[pl.ds(s * 640, 640)])
    plsc.subcore_barrier()

    def blk(b, carry):
        pltpu.sync_copy(dst_hbm.at[wid, b], idx_v)
        for k in range(IDXBLK):
            pltpu.async_copy(ones_v, acc_sh.at[idx_v.at[k]], sem_s, add=True)
            if k >= 2:
                pltpu.make_async_copy(
                    ones_v, acc_sh.at[idx_v.at[k - 2]], sem_s).wait()
        for k in (IDXBLK - 2, IDXBLK - 1):
            pltpu.make_async_copy(
                ones_v, acc_sh.at[idx_v.at[k]], sem_s).wait()
        return carry

    lax.fori_loop(0, NBLK, blk, 0)
    plsc.subcore_barrier()
    pltpu.sync_copy(
        acc_sh.at[pl.ds(s * 640, 640)],
        out_hbm.at[pl.ds(c * DEG_PAD + s * 640, 640)],
    )


@functools.partial(
    pl.kernel,
    out_type=jax.ShapeDtypeStruct((NC * ROWS_PAD, D), jnp.float32),
    mesh=_mesh,
    scratch_types=[
        pltpu.VMEM((IDXBLK, CHUNK), jnp.int32),
        pltpu.VMEM((IDXBLK, CHUNK), jnp.int32),
        [pltpu.VMEM((CHUNK, D), jnp.float32)] * 4,
        pltpu.VMEM_SHARED((ROWS_PAD, D), jnp.float32),
        [pltpu.SemaphoreType.DMA] * 4,
        [pltpu.SemaphoreType.DMA] * 4,
    ],
)
def _edge_kernel(src_hbm, dst_hbm, g1_hbm, z2d_hbm, s1_hbm,
                 idx_s, idx_d, rows, acc_sh, sg, sr):
    c = lax.axis_index("c")
    s = lax.axis_index("s")
    wid = s * NC + c

    pltpu.sync_copy(
        z2d_hbm.at[pl.ds(s * ROWS_PER_TILE, ROWS_PER_TILE)],
        acc_sh.at[pl.ds(s * ROWS_PER_TILE, ROWS_PER_TILE)],
    )
    plsc.subcore_barrier()

    def blk(b, carry):
        pltpu.sync_copy(src_hbm.at[wid, b], idx_s)
        pltpu.sync_copy(dst_hbm.at[wid, b], idx_d)
        for k in range(2):
            pltpu.async_copy(g1_hbm.at[idx_s.at[k]], rows[k], sg[k])
        for k in range(IDXBLK):
            bk = k % 4
            pltpu.make_async_copy(
                g1_hbm.at[idx_s.at[k]], rows[bk], sg[bk]).wait()
            pltpu.async_copy(
                rows[bk], acc_sh.at[idx_d.at[k]], sr[bk], add=True)
            if k + 2 < IDXBLK:
                nb = (k + 2) % 4
                if k >= 2:
                    pltpu.make_async_copy(
                        rows[nb], acc_sh.at[idx_d.at[k - 2]], sr[nb]).wait()
                pltpu.async_copy(g1_hbm.at[idx_s.at[k + 2]], rows[nb], sg[nb])
        for k in range(IDXBLK - 4, IDXBLK):
            bk = k % 4
            pltpu.make_async_copy(
                rows[bk], acc_sh.at[idx_d.at[k]], sr[bk]).wait()
        return carry

    lax.fori_loop(0, NBLK, blk, 0)
    plsc.subcore_barrier()
    pltpu.sync_copy(
        acc_sh.at[pl.ds(s * ROWS_PER_TILE, ROWS_PER_TILE)],
        s1_hbm.at[pl.ds(c * ROWS_PAD + s * ROWS_PER_TILE, ROWS_PER_TILE)],
    )


@functools.partial(
    pl.kernel,
    out_type=jax.ShapeDtypeStruct((NC * M_PAD,), jnp.float32),
    mesh=_mesh,
    scratch_types=[
        pltpu.VMEM((IDXBLK, CHUNK), jnp.int32),
        pltpu.VMEM((IDXBLK, CHUNK), jnp.int32),
        [pltpu.VMEM((CHUNK,), jnp.float32)] * 4,
        [pltpu.VMEM((CHUNK,), jnp.int32)] * 4,
        pltpu.VMEM_SHARED((M_PAD,), jnp.float32),
        [pltpu.SemaphoreType.DMA] * 4,
        [pltpu.SemaphoreType.DMA] * 4,
    ],
)
def _m_kernel(src_hbm, dst_hbm, dinv_hbm, z1d_hbm, m_hbm,
              idx_s, idx_d, dval, midx, m_sh, sv, sm):
    c = lax.axis_index("c")
    s = lax.axis_index("s")
    wid = s * NC + c

    for t in range(M_PER_TILE // M_CHUNK):
        pltpu.sync_copy(
            z1d_hbm.at[pl.ds(s * M_PER_TILE + t * M_CHUNK, M_CHUNK)],
            m_sh.at[pl.ds(s * M_PER_TILE + t * M_CHUNK, M_CHUNK)],
        )
    plsc.subcore_barrier()

    def blk(b, carry):
        pltpu.sync_copy(src_hbm.at[wid, b], idx_s)
        pltpu.sync_copy(dst_hbm.at[wid, b], idx_d)
        for k in range(2):
            pltpu.async_copy(dinv_hbm.at[idx_d.at[k]], dval[k], sv[k])
        for k in range(IDXBLK):
            bk = k % 4
            pltpu.make_async_copy(
                dinv_hbm.at[idx_d.at[k]], dval[bk], sv[bk]).wait()
            for j in range(CHUNK // 16):
                sl = pl.ds(j * 16, 16)
                d16 = idx_d[k, sl]
                s16 = idx_s[k, sl]
                grp = lax.shift_right_logical(
                    d16 * 10486, jnp.full((16,), 22, jnp.int32))
                midx[bk][sl] = grp * N + s16
            pltpu.async_copy(dval[bk], m_sh.at[midx[bk]], sm[bk], add=True)
            if k + 2 < IDXBLK:
                nb = (k + 2) % 4
                if k >= 2:
                    pltpu.make_async_copy(
                        dval[nb], m_sh.at[midx[nb]], sm[nb]).wait()
                pltpu.async_copy(dinv_hbm.at[idx_d.at[k + 2]], dval[nb], sv[nb])
        for k in range(IDXBLK - 4, IDXBLK):
            bk = k % 4
            pltpu.make_async_copy(
                dval[bk], m_sh.at[midx[bk]], sm[bk]).wait()
        return carry

    lax.fori_loop(0, NBLK, blk, 0)
    plsc.subcore_barrier()
    for t in range(M_PER_TILE // M_CHUNK):
        pltpu.sync_copy(
            m_sh.at[pl.ds(s * M_PER_TILE + t * M_CHUNK, M_CHUNK)],
            m_hbm.at[pl.ds(c * M_PAD + s * M_PER_TILE + t * M_CHUNK, M_CHUNK)],
        )


_PRE_BLK = 2000
_PRE_GRID = N // _PRE_BLK


def _tc_pre_body(x_ref, w1_ref, dp0b_ref, dp1b_ref, dp0f_ref, dp1f_ref,
                 g1_ref, hself_ref, dinv_ref, invdeg_ref):
    i = pl.program_id(0)
    h1 = jnp.dot(x_ref[...], w1_ref[...],
                 preferred_element_type=jnp.float32, precision=_HIGH)
    degb = dp0b_ref[...] + dp1b_ref[...] + 1.0
    g1_ref[...] = h1 * (1.0 / jnp.sqrt(degb))
    hself_ref[...] = h1 * (1.0 / degb)

    @pl.when(i == 0)
    def _():
        degf = dp0f_ref[...] + dp1f_ref[...] + 1.0
        dinv_ref[...] = 1.0 / jnp.sqrt(degf)
        invdeg_ref[...] = 1.0 / degf


_tc_pre = pl.pallas_call(
    _tc_pre_body,
    grid=(_PRE_GRID,),
    in_specs=[
        pl.BlockSpec((_PRE_BLK, D), lambda i: (i, 0)),
        pl.BlockSpec((D, D), lambda i: (0, 0)),
        pl.BlockSpec((_PRE_BLK, 1), lambda i: (i, 0)),
        pl.BlockSpec((_PRE_BLK, 1), lambda i: (i, 0)),
        pl.BlockSpec((DEG_PAD, 1), lambda i: (0, 0)),
        pl.BlockSpec((DEG_PAD, 1), lambda i: (0, 0)),
    ],
    out_specs=(
        pl.BlockSpec((_PRE_BLK, D), lambda i: (i, 0)),
        pl.BlockSpec((_PRE_BLK, D), lambda i: (i, 0)),
        pl.BlockSpec((DEG_PAD, 1), lambda i: (0, 0)),
        pl.BlockSpec((DEG_PAD, 1), lambda i: (0, 0)),
    ),
    out_shape=(
        jax.ShapeDtypeStruct((N, D), jnp.float32),
        jax.ShapeDtypeStruct((N, D), jnp.float32),
        jax.ShapeDtypeStruct((DEG_PAD, 1), jnp.float32),
        jax.ShapeDtypeStruct((DEG_PAD, 1), jnp.float32),
    ),
)


def _tc_post_body(s0_ref, s1_ref, hself_ref, dinv_ref, invdeg_ref,
                  m0_ref, m1_ref, w2_ref, b1_ref, b2_ref, out_ref):
    dinv = dinv_ref[...][:N]
    invdeg = invdeg_ref[...][:N]
    s1 = s0_ref[...] + s1_ref[...]
    hr = jnp.maximum(dinv * s1 + hself_ref[...] + b1_ref[...], 0.0)
    t = dinv * hr
    u = invdeg * hr
    m = m0_ref[...] + m1_ref[...]
    gi = lax.broadcasted_iota(jnp.int32, (NGRP, N), 0)
    vi = lax.broadcasted_iota(jnp.int32, (NGRP, N), 1)
    diff = vi - gi * GRP
    p = jnp.where((diff >= 0) & (diff < GRP), 1.0, 0.0).astype(jnp.float32)
    agg = (jnp.dot(m, t, preferred_element_type=jnp.float32, precision=_HIGH)
           + jnp.dot(p, u, preferred_element_type=jnp.float32,
                     precision=_HIGH))
    out = jnp.dot(agg, w2_ref[...], preferred_element_type=jnp.float32,
                  precision=_HIGH)
    out_ref[...] = out * (1.0 / GRP) + b2_ref[...]


_tc_post = pl.pallas_call(
    _tc_post_body,
    out_shape=jax.ShapeDtypeStruct((NGRP, 40), jnp.float32),
)


def kernel(x, edge_index, W1, b1, W2, b2):
    src = edge_index[0].reshape(NW, NBLK, IDXBLK, CHUNK)
    dst3 = edge_index[1].reshape(NW, NBLK, IDXBLK, CHUNK)

    degp = _deg_kernel(dst3)
    dp = degp.reshape(NC, DEG_PAD, 1)
    g1, hself, dinv2d, invdeg2d = _tc_pre(x, W1, dp[0], dp[1], dp[0], dp[1])

    z2d = jnp.zeros((ROWS_PAD, D), jnp.float32)
    z1d = jnp.zeros((M_PAD,), jnp.float32)
    s1p = _edge_kernel(src, dst3, g1, z2d)
    mp = _m_kernel(src, dst3, dinv2d.reshape(DEG_PAD), z1d)

    s1p = s1p.reshape(NC, ROWS_PAD, D)
    mp = mp.reshape(NC, M_PAD)
    m0 = mp[0, : NGRP * N].reshape(NGRP, N)
    m1 = mp[1, : NGRP * N].reshape(NGRP, N)

    out = _tc_post(s1p[0, :N], s1p[1, :N], hself, dinv2d, invdeg2d,
                   m0, m1, W2, b1.reshape(1, D), b2.reshape(1, 40))
    return out


# speedup vs baseline: 33.8103x; 1.1437x over previous
import functools

import jax
import jax.numpy as jnp
from jax import lax
from jax.experimental import pallas as pl
from jax.experimental.pallas import tpu as pltpu
import jax.experimental.pallas.tpu_sc as plsc

N = 10000
E = 320000
D = 128
NGRP = 25
GRP = 400

NC = 2
NS = 16
NW = NC * NS
EPW = E // NW
CHUNK = 80
NCHUNK = EPW // CHUNK
IDXBLK = 25
NBLK = NCHUNK // IDXBLK

DEG_PAD = 10240
M_PER_TILE = 16000
M_CHUNK = 640
M_PAD = M_PER_TILE * NS
ROWS_PER_TILE = 632
ROWS_PAD = ROWS_PER_TILE * NS

_HIGH = jax.lax.Precision.HIGHEST

_mesh = plsc.VectorSubcoreMesh(core_axis_name="c", subcore_axis_name="s")


@functools.partial(
    pl.kernel,
    out_type=jax.ShapeDtypeStruct((NC * DEG_PAD,), jnp.float32),
    mesh=_mesh,
    scratch_types=[
        pltpu.VMEM((IDXBLK, CHUNK), jnp.int32),
        pltpu.VMEM((CHUNK,), jnp.float32),
        pltpu.VMEM((640,), jnp.float32),
        pltpu.VMEM_SHARED((DEG_PAD,), jnp.float32),
        pltpu.SemaphoreType.DMA,
    ],
)
def _deg_kernel(dst_hbm, out_hbm, idx_v, ones_v, zero_v, acc_sh, sem_s):
    c = lax.axis_index("c")
    s = lax.axis_index("s")
    wid = s * NC + c

    for j in range(CHUNK // 16):
        ones_v[pl.ds(j * 16, 16)] = jnp.full((16,), 1.0, jnp.float32)
    for j in range(640 // 16):
        zero_v[pl.ds(j * 16, 16)] = jnp.zeros((16,), jnp.float32)
    pltpu.sync_copy(zero_v, acc_sh.at[pl.ds(s * 640, 640)])
    plsc.subcore_barrier()

    def blk(b, carry):
        pltpu.sync_copy(dst_hbm.at[wid, b], idx_v)
        for k in range(IDXBLK):
            pltpu.async_copy(ones_v, acc_sh.at[idx_v.at[k]], sem_s, add=True)
            if k >= 2:
                pltpu.make_async_copy(
                    ones_v, acc_sh.at[idx_v.at[k - 2]], sem_s).wait()
        for k in (IDXBLK - 2, IDXBLK - 1):
            pltpu.make_async_copy(
                ones_v, acc_sh.at[idx_v.at[k]], sem_s).wait()
        return carry

    lax.fori_loop(0, NBLK, blk, 0)
    plsc.subcore_barrier()
    pltpu.sync_copy(
        acc_sh.at[pl.ds(s * 640, 640)],
        out_hbm.at[pl.ds(c * DEG_PAD + s * 640, 640)],
    )


@functools.partial(
    pl.kernel,
    out_type=jax.ShapeDtypeStruct((NC * ROWS_PAD, D), jnp.float32),
    mesh=_mesh,
    scratch_types=[
        pltpu.VMEM((IDXBLK, CHUNK), jnp.int32),
        pltpu.VMEM((IDXBLK, CHUNK), jnp.int32),
        [pltpu.VMEM((CHUNK, D), jnp.float32)] * 4,
        pltpu.VMEM_SHARED((ROWS_PAD, D), jnp.float32),
        [pltpu.SemaphoreType.DMA] * 4,
        [pltpu.SemaphoreType.DMA] * 4,
    ],
)
def _edge_kernel(src_hbm, dst_hbm, g1_hbm, z2d_hbm, s1_hbm,
                 idx_s, idx_d, rows, acc_sh, sg, sr):
    c = lax.axis_index("c")
    s = lax.axis_index("s")
    wid = s * NC + c

    pltpu.sync_copy(
        z2d_hbm.at[pl.ds(s * ROWS_PER_TILE, ROWS_PER_TILE)],
        acc_sh.at[pl.ds(s * ROWS_PER_TILE, ROWS_PER_TILE)],
    )
    plsc.subcore_barrier()

    def blk(b, carry):
        pltpu.sync_copy(src_hbm.at[wid, b], idx_s)
        pltpu.sync_copy(dst_hbm.at[wid, b], idx_d)
        for k in range(2):
            pltpu.async_copy(g1_hbm.at[idx_s.at[k]], rows[k], sg[k])
        for k in range(IDXBLK):
            bk = k % 4
            pltpu.make_async_copy(
                g1_hbm.at[idx_s.at[k]], rows[bk], sg[bk]).wait()
            pltpu.async_copy(
                rows[bk], acc_sh.at[idx_d.at[k]], sr[bk], add=True)
            if k + 2 < IDXBLK:
                nb = (k + 2) % 4
                if k >= 2:
                    pltpu.make_async_copy(
                        rows[nb], acc_sh.at[idx_d.at[k - 2]], sr[nb]).wait()
                pltpu.async_copy(g1_hbm.at[idx_s.at[k + 2]], rows[nb], sg[nb])
        for k in range(IDXBLK - 4, IDXBLK):
            bk = k % 4
            pltpu.make_async_copy(
                rows[bk], acc_sh.at[idx_d.at[k]], sr[bk]).wait()
        return carry

    lax.fori_loop(0, NBLK, blk, 0)
    plsc.subcore_barrier()
    pltpu.sync_copy(
        acc_sh.at[pl.ds(s * ROWS_PER_TILE, ROWS_PER_TILE)],
        s1_hbm.at[pl.ds(c * ROWS_PAD + s * ROWS_PER_TILE, ROWS_PER_TILE)],
    )


@functools.partial(
    pl.kernel,
    out_type=jax.ShapeDtypeStruct((NC * M_PAD,), jnp.float32),
    mesh=_mesh,
    scratch_types=[
        pltpu.VMEM((IDXBLK, CHUNK), jnp.int32),
        pltpu.VMEM((IDXBLK, CHUNK), jnp.int32),
        [pltpu.VMEM((CHUNK,), jnp.float32)] * 4,
        [pltpu.VMEM((CHUNK,), jnp.int32)] * 4,
        pltpu.VMEM_SHARED((M_PAD,), jnp.float32),
        pltpu.VMEM_SHARED((DEG_PAD,), jnp.float32),
        [pltpu.SemaphoreType.DMA] * 4,
        [pltpu.SemaphoreType.DMA] * 4,
    ],
)
def _m_kernel(src_hbm, dst_hbm, dinv_hbm, z1d_hbm, m_hbm,
              idx_s, idx_d, dval, midx, m_sh, dinv_sh, sv, sm):
    c = lax.axis_index("c")
    s = lax.axis_index("s")
    wid = s * NC + c

    pltpu.sync_copy(
        dinv_hbm.at[pl.ds(s * 640, 640)],
        dinv_sh.at[pl.ds(s * 640, 640)],
    )
    for t in range(M_PER_TILE // M_CHUNK):
        pltpu.sync_copy(
            z1d_hbm.at[pl.ds(s * M_PER_TILE + t * M_CHUNK, M_CHUNK)],
            m_sh.at[pl.ds(s * M_PER_TILE + t * M_CHUNK, M_CHUNK)],
        )
    plsc.subcore_barrier()

    def blk(b, carry):
        pltpu.sync_copy(src_hbm.at[wid, b], idx_s)
        pltpu.sync_copy(dst_hbm.at[wid, b], idx_d)
        for k in range(2):
            pltpu.async_copy(dinv_sh.at[idx_d.at[k]], dval[k], sv[k])
        for k in range(IDXBLK):
            bk = k % 4
            pltpu.make_async_copy(
                dinv_sh.at[idx_d.at[k]], dval[bk], sv[bk]).wait()
            for j in range(CHUNK // 16):
                sl = pl.ds(j * 16, 16)
                d16 = idx_d[k, sl]
                s16 = idx_s[k, sl]
                grp = lax.shift_right_logical(
                    d16 * 10486, jnp.full((16,), 22, jnp.int32))
                midx[bk][sl] = grp * N + s16
            pltpu.async_copy(dval[bk], m_sh.at[midx[bk]], sm[bk], add=True)
            if k + 2 < IDXBLK:
                nb = (k + 2) % 4
                if k >= 2:
                    pltpu.make_async_copy(
                        dval[nb], m_sh.at[midx[nb]], sm[nb]).wait()
                pltpu.async_copy(dinv_sh.at[idx_d.at[k + 2]], dval[nb], sv[nb])
        for k in range(IDXBLK - 4, IDXBLK):
            bk = k % 4
            pltpu.make_async_copy(
                dval[bk], m_sh.at[midx[bk]], sm[bk]).wait()
        return carry

    lax.fori_loop(0, NBLK, blk, 0)
    plsc.subcore_barrier()
    for t in range(M_PER_TILE // M_CHUNK):
        pltpu.sync_copy(
            m_sh.at[pl.ds(s * M_PER_TILE + t * M_CHUNK, M_CHUNK)],
            m_hbm.at[pl.ds(c * M_PAD + s * M_PER_TILE + t * M_CHUNK, M_CHUNK)],
        )


_PRE_BLK = 2000
_PRE_GRID = N // _PRE_BLK


def _tc_pre_body(x_ref, w1_ref, dp0b_ref, dp1b_ref, dp0f_ref, dp1f_ref,
                 g1_ref, hself_ref, dinv_ref, invdeg_ref):
    i = pl.program_id(0)
    h1 = jnp.dot(x_ref[...], w1_ref[...],
                 preferred_element_type=jnp.float32, precision=_HIGH)
    degb = dp0b_ref[...] + dp1b_ref[...] + 1.0
    g1_ref[...] = h1 * (1.0 / jnp.sqrt(degb))
    hself_ref[...] = h1 * (1.0 / degb)

    @pl.when(i == 0)
    def _():
        degf = dp0f_ref[...] + dp1f_ref[...] + 1.0
        dinv_ref[...] = 1.0 / jnp.sqrt(degf)
        invdeg_ref[...] = 1.0 / degf


_tc_pre = pl.pallas_call(
    _tc_pre_body,
    grid=(_PRE_GRID,),
    in_specs=[
        pl.BlockSpec((_PRE_BLK, D), lambda i: (i, 0)),
        pl.BlockSpec((D, D), lambda i: (0, 0)),
        pl.BlockSpec((_PRE_BLK, 1), lambda i: (i, 0)),
        pl.BlockSpec((_PRE_BLK, 1), lambda i: (i, 0)),
        pl.BlockSpec((DEG_PAD, 1), lambda i: (0, 0)),
        pl.BlockSpec((DEG_PAD, 1), lambda i: (0, 0)),
    ],
    out_specs=(
        pl.BlockSpec((_PRE_BLK, D), lambda i: (i, 0)),
        pl.BlockSpec((_PRE_BLK, D), lambda i: (i, 0)),
        pl.BlockSpec((DEG_PAD, 1), lambda i: (0, 0)),
        pl.BlockSpec((DEG_PAD, 1), lambda i: (0, 0)),
    ),
    out_shape=(
        jax.ShapeDtypeStruct((N, D), jnp.float32),
        jax.ShapeDtypeStruct((N, D), jnp.float32),
        jax.ShapeDtypeStruct((DEG_PAD, 1), jnp.float32),
        jax.ShapeDtypeStruct((DEG_PAD, 1), jnp.float32),
    ),
)


def _tc_post_body(s0_ref, s1_ref, hself_ref, dinv_ref, invdeg_ref,
                  m0_ref, m1_ref, w2_ref, b1_ref, b2_ref, out_ref):
    dinv = dinv_ref[...][:N]
    invdeg = invdeg_ref[...][:N]
    s1 = s0_ref[...] + s1_ref[...]
    hr = jnp.maximum(dinv * s1 + hself_ref[...] + b1_ref[...], 0.0)
    t = dinv * hr
    u = invdeg * hr
    m = m0_ref[...] + m1_ref[...]
    gi = lax.broadcasted_iota(jnp.int32, (NGRP, N), 0)
    vi = lax.broadcasted_iota(jnp.int32, (NGRP, N), 1)
    diff = vi - gi * GRP
    p = jnp.where((diff >= 0) & (diff < GRP), 1.0, 0.0).astype(jnp.float32)
    agg = (jnp.dot(m, t, preferred_element_type=jnp.float32, precision=_HIGH)
           + jnp.dot(p, u, preferred_element_type=jnp.float32,
                     precision=_HIGH))
    out = jnp.dot(agg, w2_ref[...], preferred_element_type=jnp.float32,
                  precision=_HIGH)
    out_ref[...] = out * (1.0 / GRP) + b2_ref[...]


_tc_post = pl.pallas_call(
    _tc_post_body,
    out_shape=jax.ShapeDtypeStruct((NGRP, 40), jnp.float32),
)


def kernel(x, edge_index, W1, b1, W2, b2):
    src = edge_index[0].reshape(NW, NBLK, IDXBLK, CHUNK)
    dst3 = edge_index[1].reshape(NW, NBLK, IDXBLK, CHUNK)

    degp = _deg_kernel(dst3)
    dp = degp.reshape(NC, DEG_PAD, 1)
    g1, hself, dinv2d, invdeg2d = _tc_pre(x, W1, dp[0], dp[1], dp[0], dp[1])

    z2d = jnp.zeros((ROWS_PAD, D), jnp.float32)
    z1d = jnp.zeros((M_PAD,), jnp.float32)
    s1p = _edge_kernel(src, dst3, g1, z2d)
    mp = _m_kernel(src, dst3, dinv2d.reshape(DEG_PAD), z1d)

    s1p = s1p.reshape(NC, ROWS_PAD, D)
    mp = mp.reshape(NC, M_PAD)
    m0 = mp[0, : NGRP * N].reshape(NGRP, N)
    m1 = mp[1, : NGRP * N].reshape(NGRP, N)

    out = _tc_post(s1p[0, :N], s1p[1, :N], hself, dinv2d, invdeg2d,
                   m0, m1, W2, b1.reshape(1, D), b2.reshape(1, 40))
    return out


# speedup vs baseline: 35.3235x; 1.0448x over previous
import functools

import jax
import jax.numpy as jnp
from jax import lax
from jax.experimental import pallas as pl
from jax.experimental.pallas import tpu as pltpu
import jax.experimental.pallas.tpu_sc as plsc

N = 10000
E = 320000
D = 128
NGRP = 25
GRP = 400

NC = 2
NS = 16
NW = NC * NS
EPW = E // NW
CHUNK = 80
NCHUNK = EPW // CHUNK
IDXBLK = 25
NBLK = NCHUNK // IDXBLK

DEG_PAD = 10240
M_PER_TILE = 16000
M_CHUNK = 640
M_PAD = M_PER_TILE * NS
ROWS_PER_TILE = 632
ROWS_PAD = ROWS_PER_TILE * NS

_HIGH = jax.lax.Precision.HIGHEST

_mesh = plsc.VectorSubcoreMesh(core_axis_name="c", subcore_axis_name="s")


@functools.partial(
    pl.kernel,
    out_type=jax.ShapeDtypeStruct((NC * DEG_PAD,), jnp.float32),
    mesh=_mesh,
    scratch_types=[
        pltpu.VMEM((IDXBLK, CHUNK), jnp.int32),
        pltpu.VMEM((CHUNK,), jnp.float32),
        pltpu.VMEM((640,), jnp.float32),
        pltpu.VMEM_SHARED((DEG_PAD,), jnp.float32),
        pltpu.SemaphoreType.DMA,
    ],
)
def _deg_kernel(dst_hbm, out_hbm, idx_v, ones_v, zero_v, acc_sh, sem_s):
    c = lax.axis_index("c")
    s = lax.axis_index("s")
    wid = s * NC + c

    for j in range(CHUNK // 16):
        ones_v[pl.ds(j * 16, 16)] = jnp.full((16,), 1.0, jnp.float32)
    for j in range(640 // 16):
        zero_v[pl.ds(j * 16, 16)] = jnp.zeros((16,), jnp.float32)
    pltpu.sync_copy(zero_v, acc_sh.at[pl.ds(s * 640, 640)])
    plsc.subcore_barrier()

    def blk(b, carry):
        pltpu.sync_copy(dst_hbm.at[wid, b], idx_v)
        for k in range(IDXBLK):
            pltpu.async_copy(ones_v, acc_sh.at[idx_v.at[k]], sem_s, add=True)
            if k >= 2:
                pltpu.make_async_copy(
                    ones_v, acc_sh.at[idx_v.at[k - 2]], sem_s).wait()
        for k in (IDXBLK - 2, IDXBLK - 1):
            pltpu.make_async_copy(
                ones_v, acc_sh.at[idx_v.at[k]], sem_s).wait()
        return carry

    lax.fori_loop(0, NBLK, blk, 0)
    plsc.subcore_barrier()
    pltpu.sync_copy(
        acc_sh.at[pl.ds(s * 640, 640)],
        out_hbm.at[pl.ds(c * DEG_PAD + s * 640, 640)],
    )


@functools.partial(
    pl.kernel,
    out_type=jax.ShapeDtypeStruct((NC * ROWS_PAD, D), jnp.float32),
    mesh=_mesh,
    scratch_types=[
        pltpu.VMEM((IDXBLK, CHUNK), jnp.int32),
        pltpu.VMEM((IDXBLK, CHUNK), jnp.int32),
        [pltpu.VMEM((CHUNK, D), jnp.float32)] * 4,
        pltpu.VMEM_SHARED((ROWS_PAD, D), jnp.float32),
        [pltpu.SemaphoreType.DMA] * 4,
        [pltpu.SemaphoreType.DMA] * 4,
    ],
)
def _edge_kernel(src_hbm, dst_hbm, g1_hbm, z2d_hbm, s1_hbm,
                 idx_s, idx_d, rows, acc_sh, sg, sr):
    c = lax.axis_index("c")
    s = lax.axis_index("s")
    wid = s * NC + c

    pltpu.sync_copy(
        z2d_hbm.at[pl.ds(s * ROWS_PER_TILE, ROWS_PER_TILE)],
        acc_sh.at[pl.ds(s * ROWS_PER_TILE, ROWS_PER_TILE)],
    )
    plsc.subcore_barrier()

    def blk(b, carry):
        pltpu.sync_copy(src_hbm.at[wid, b], idx_s)
        pltpu.sync_copy(dst_hbm.at[wid, b], idx_d)
        for k in range(3):
            pltpu.async_copy(g1_hbm.at[idx_s.at[k]], rows[k], sg[k])
        for k in range(IDXBLK):
            bk = k % 4
            pltpu.make_async_copy(
                g1_hbm.at[idx_s.at[k]], rows[bk], sg[bk]).wait()
            pltpu.async_copy(
                rows[bk], acc_sh.at[idx_d.at[k]], sr[bk], add=True)
            if k + 3 < IDXBLK:
                nb = (k + 3) % 4
                if k >= 1:
                    pltpu.make_async_copy(
                        rows[nb], acc_sh.at[idx_d.at[k - 1]], sr[nb]).wait()
                pltpu.async_copy(g1_hbm.at[idx_s.at[k + 3]], rows[nb], sg[nb])
        for k in range(IDXBLK - 4, IDXBLK):
            bk = k % 4
            pltpu.make_async_copy(
                rows[bk], acc_sh.at[idx_d.at[k]], sr[bk]).wait()
        return carry

    lax.fori_loop(0, NBLK, blk, 0)
    plsc.subcore_barrier()
    pltpu.sync_copy(
        acc_sh.at[pl.ds(s * ROWS_PER_TILE, ROWS_PER_TILE)],
        s1_hbm.at[pl.ds(c * ROWS_PAD + s * ROWS_PER_TILE, ROWS_PER_TILE)],
    )


@functools.partial(
    pl.kernel,
    out_type=jax.ShapeDtypeStruct((NC * M_PAD,), jnp.float32),
    mesh=_mesh,
    scratch_types=[
        pltpu.VMEM((IDXBLK, CHUNK), jnp.int32),
        pltpu.VMEM((IDXBLK, CHUNK), jnp.int32),
        [pltpu.VMEM((CHUNK,), jnp.float32)] * 4,
        [pltpu.VMEM((CHUNK,), jnp.int32)] * 4,
        pltpu.VMEM_SHARED((M_PAD,), jnp.float32),
        pltpu.VMEM_SHARED((DEG_PAD,), jnp.float32),
        [pltpu.SemaphoreType.DMA] * 4,
        [pltpu.SemaphoreType.DMA] * 4,
    ],
)
def _m_kernel(src_hbm, dst_hbm, dinv_hbm, z1d_hbm, m_hbm,
              idx_s, idx_d, dval, midx, m_sh, dinv_sh, sv, sm):
    c = lax.axis_index("c")
    s = lax.axis_index("s")
    wid = s * NC + c

    pltpu.sync_copy(
        dinv_hbm.at[pl.ds(s * 640, 640)],
        dinv_sh.at[pl.ds(s * 640, 640)],
    )
    for t in range(M_PER_TILE // M_CHUNK):
        pltpu.sync_copy(
            z1d_hbm.at[pl.ds(s * M_PER_TILE + t * M_CHUNK, M_CHUNK)],
            m_sh.at[pl.ds(s * M_PER_TILE + t * M_CHUNK, M_CHUNK)],
        )
    plsc.subcore_barrier()

    def blk(b, carry):
        pltpu.sync_copy(src_hbm.at[wid, b], idx_s)
        pltpu.sync_copy(dst_hbm.at[wid, b], idx_d)
        for k in range(2):
            pltpu.async_copy(dinv_sh.at[idx_d.at[k]], dval[k], sv[k])
        for k in range(IDXBLK):
            bk = k % 4
            pltpu.make_async_copy(
                dinv_sh.at[idx_d.at[k]], dval[bk], sv[bk]).wait()
            for j in range(CHUNK // 16):
                sl = pl.ds(j * 16, 16)
                d16 = idx_d[k, sl]
                s16 = idx_s[k, sl]
                grp = lax.shift_right_logical(
                    d16 * 10486, jnp.full((16,), 22, jnp.int32))
                midx[bk][sl] = grp * N + s16
            pltpu.async_copy(dval[bk], m_sh.at[midx[bk]], sm[bk], add=True)
            if k + 2 < IDXBLK:
                nb = (k + 2) % 4
                if k >= 2:
                    pltpu.make_async_copy(
                        dval[nb], m_sh.at[midx[nb]], sm[nb]).wait()
                pltpu.async_copy(dinv_sh.at[idx_d.at[k + 2]], dval[nb], sv[nb])
        for k in range(IDXBLK - 4, IDXBLK):
            bk = k % 4
            pltpu.make_async_copy(
                dval[bk], m_sh.at[midx[bk]], sm[bk]).wait()
        return carry

    lax.fori_loop(0, NBLK, blk, 0)
    plsc.subcore_barrier()
    for t in range(M_PER_TILE // M_CHUNK):
        pltpu.sync_copy(
            m_sh.at[pl.ds(s * M_PER_TILE + t * M_CHUNK, M_CHUNK)],
            m_hbm.at[pl.ds(c * M_PAD + s * M_PER_TILE + t * M_CHUNK, M_CHUNK)],
        )


_PRE_BLK = 2000
_PRE_GRID = N // _PRE_BLK


def _tc_pre_body(x_ref, w1_ref, dp0b_ref, dp1b_ref, dp0f_ref, dp1f_ref,
                 g1_ref, hself_ref, dinv_ref, invdeg_ref):
    i = pl.program_id(0)
    h1 = jnp.dot(x_ref[...], w1_ref[...],
                 preferred_element_type=jnp.float32, precision=_HIGH)
    degb = dp0b_ref[...] + dp1b_ref[...] + 1.0
    g1_ref[...] = h1 * (1.0 / jnp.sqrt(degb))
    hself_ref[...] = h1 * (1.0 / degb)

    @pl.when(i == 0)
    def _():
        degf = dp0f_ref[...] + dp1f_ref[...] + 1.0
        dinv_ref[...] = 1.0 / jnp.sqrt(degf)
        invdeg_ref[...] = 1.0 / degf


_tc_pre = pl.pallas_call(
    _tc_pre_body,
    grid=(_PRE_GRID,),
    in_specs=[
        pl.BlockSpec((_PRE_BLK, D), lambda i: (i, 0)),
        pl.BlockSpec((D, D), lambda i: (0, 0)),
        pl.BlockSpec((_PRE_BLK, 1), lambda i: (i, 0)),
        pl.BlockSpec((_PRE_BLK, 1), lambda i: (i, 0)),
        pl.BlockSpec((DEG_PAD, 1), lambda i: (0, 0)),
        pl.BlockSpec((DEG_PAD, 1), lambda i: (0, 0)),
    ],
    out_specs=(
        pl.BlockSpec((_PRE_BLK, D), lambda i: (i, 0)),
        pl.BlockSpec((_PRE_BLK, D), lambda i: (i, 0)),
        pl.BlockSpec((DEG_PAD, 1), lambda i: (0, 0)),
        pl.BlockSpec((DEG_PAD, 1), lambda i: (0, 0)),
    ),
    out_shape=(
        jax.ShapeDtypeStruct((N, D), jnp.float32),
        jax.ShapeDtypeStruct((N, D), jnp.float32),
        jax.ShapeDtypeStruct((DEG_PAD, 1), jnp.float32),
        jax.ShapeDtypeStruct((DEG_PAD, 1), jnp.float32),
    ),
)


def _tc_post_body(s0_ref, s1_ref, hself_ref, dinv_ref, invdeg_ref,
                  m0_ref, m1_ref, w2_ref, b1_ref, b2_ref, out_ref):
    dinv = dinv_ref[...][:N]
    invdeg = invdeg_ref[...][:N]
    s1 = s0_ref[...] + s1_ref[...]
    hr = jnp.maximum(dinv * s1 + hself_ref[...] + b1_ref[...], 0.0)
    t = dinv * hr
    u = invdeg * hr
    m = m0_ref[...] + m1_ref[...]
    agg = (jnp.dot(m, t, preferred_element_type=jnp.float32, precision=_HIGH)
           + jnp.sum(u.reshape(NGRP, GRP, D), axis=1))
    out = jnp.dot(agg, w2_ref[...], preferred_element_type=jnp.float32,
                  precision=_HIGH)
    out_ref[...] = out * (1.0 / GRP) + b2_ref[...]


_tc_post = pl.pallas_call(
    _tc_post_body,
    out_shape=jax.ShapeDtypeStruct((NGRP, 40), jnp.float32),
)


def kernel(x, edge_index, W1, b1, W2, b2):
    src = edge_index[0].reshape(NW, NBLK, IDXBLK, CHUNK)
    dst3 = edge_index[1].reshape(NW, NBLK, IDXBLK, CHUNK)

    degp = _deg_kernel(dst3)
    dp = degp.reshape(NC, DEG_PAD, 1)
    g1, hself, dinv2d, invdeg2d = _tc_pre(x, W1, dp[0], dp[1], dp[0], dp[1])

    z2d = jnp.zeros((ROWS_PAD, D), jnp.float32)
    z1d = jnp.zeros((M_PAD,), jnp.float32)
    s1p = _edge_kernel(src, dst3, g1, z2d)
    mp = _m_kernel(src, dst3, dinv2d.reshape(DEG_PAD), z1d)

    s1p = s1p.reshape(NC, ROWS_PAD, D)
    mp = mp.reshape(NC, M_PAD)
    m0 = mp[0, : NGRP * N].reshape(NGRP, N)
    m1 = mp[1, : NGRP * N].reshape(NGRP, N)

    out = _tc_post(s1p[0, :N], s1p[1, :N], hself, dinv2d, invdeg2d,
                   m0, m1, W2, b1.reshape(1, D), b2.reshape(1, 40))
    return out


# speedup vs baseline: 35.6630x; 1.0096x over previous
import functools

import jax
import jax.numpy as jnp
from jax import lax
from jax.experimental import pallas as pl
from jax.experimental.pallas import tpu as pltpu
import jax.experimental.pallas.tpu_sc as plsc

N = 10000
E = 320000
D = 128
NGRP = 25
GRP = 400

NC = 2
NS = 16
NW = NC * NS
EPW = E // NW
CHUNK = 80
NCHUNK = EPW // CHUNK
IDXBLK = 25
NBLK = NCHUNK // IDXBLK

DEG_PAD = 10240
M_PER_TILE = 16000
M_CHUNK = 640
M_PAD = M_PER_TILE * NS
ROWS_PER_TILE = 632
ROWS_PAD = ROWS_PER_TILE * NS

_HIGH = jax.lax.Precision.HIGHEST

_mesh = plsc.VectorSubcoreMesh(core_axis_name="c", subcore_axis_name="s")


@functools.partial(
    pl.kernel,
    out_type=jax.ShapeDtypeStruct((NC * DEG_PAD,), jnp.float32),
    mesh=_mesh,
    scratch_types=[
        pltpu.VMEM((IDXBLK, CHUNK), jnp.int32),
        pltpu.VMEM((CHUNK,), jnp.float32),
        pltpu.VMEM((640,), jnp.float32),
        pltpu.VMEM_SHARED((DEG_PAD,), jnp.float32),
        pltpu.SemaphoreType.DMA,
    ],
)
def _deg_kernel(dst_hbm, out_hbm, idx_v, ones_v, zero_v, acc_sh, sem_s):
    c = lax.axis_index("c")
    s = lax.axis_index("s")
    wid = s * NC + c

    for j in range(CHUNK // 16):
        ones_v[pl.ds(j * 16, 16)] = jnp.full((16,), 1.0, jnp.float32)
    for j in range(640 // 16):
        zero_v[pl.ds(j * 16, 16)] = jnp.zeros((16,), jnp.float32)
    pltpu.sync_copy(zero_v, acc_sh.at[pl.ds(s * 640, 640)])
    plsc.subcore_barrier()

    def blk(b, carry):
        pltpu.sync_copy(dst_hbm.at[wid, b], idx_v)
        for k in range(IDXBLK):
            pltpu.async_copy(ones_v, acc_sh.at[idx_v.at[k]], sem_s, add=True)
            if k >= 2:
                pltpu.make_async_copy(
                    ones_v, acc_sh.at[idx_v.at[k - 2]], sem_s).wait()
        for k in (IDXBLK - 2, IDXBLK - 1):
            pltpu.make_async_copy(
                ones_v, acc_sh.at[idx_v.at[k]], sem_s).wait()
        return carry

    lax.fori_loop(0, NBLK, blk, 0)
    plsc.subcore_barrier()
    pltpu.sync_copy(
        acc_sh.at[pl.ds(s * 640, 640)],
        out_hbm.at[pl.ds(c * DEG_PAD + s * 640, 640)],
    )


@functools.partial(
    pl.kernel,
    out_type=jax.ShapeDtypeStruct((NC * ROWS_PAD, D), jnp.float32),
    mesh=_mesh,
    scratch_types=[
        pltpu.VMEM((IDXBLK, CHUNK), jnp.int32),
        pltpu.VMEM((IDXBLK, CHUNK), jnp.int32),
        [pltpu.VMEM((CHUNK, D), jnp.float32)] * 4,
        pltpu.VMEM_SHARED((ROWS_PAD, D), jnp.float32),
        [pltpu.SemaphoreType.DMA] * 4,
        [pltpu.SemaphoreType.DMA] * 4,
    ],
)
def _edge_kernel(src_hbm, dst_hbm, g1_hbm, z2d_hbm, s1_hbm,
                 idx_s, idx_d, rows, acc_sh, sg, sr):
    c = lax.axis_index("c")
    s = lax.axis_index("s")
    wid = s * NC + c

    pltpu.sync_copy(
        z2d_hbm.at[pl.ds(s * ROWS_PER_TILE, ROWS_PER_TILE)],
        acc_sh.at[pl.ds(s * ROWS_PER_TILE, ROWS_PER_TILE)],
    )
    plsc.subcore_barrier()

    def blk(b, carry):
        pltpu.sync_copy(src_hbm.at[wid, b], idx_s)
        pltpu.sync_copy(dst_hbm.at[wid, b], idx_d)
        for k in range(3):
            pltpu.async_copy(g1_hbm.at[idx_s.at[k]], rows[k], sg[k])
        for k in range(IDXBLK):
            bk = k % 4
            pltpu.make_async_copy(
                g1_hbm.at[idx_s.at[k]], rows[bk], sg[bk]).wait()
            pltpu.async_copy(
                rows[bk], acc_sh.at[idx_d.at[k]], sr[bk], add=True)
            if k + 3 < IDXBLK:
                nb = (k + 3) % 4
                if k >= 1:
                    pltpu.make_async_copy(
                        rows[nb], acc_sh.at[idx_d.at[k - 1]], sr[nb]).wait()
                pltpu.async_copy(g1_hbm.at[idx_s.at[k + 3]], rows[nb], sg[nb])
        for k in range(IDXBLK - 4, IDXBLK):
            bk = k % 4
            pltpu.make_async_copy(
                rows[bk], acc_sh.at[idx_d.at[k]], sr[bk]).wait()
        return carry

    lax.fori_loop(0, NBLK, blk, 0)
    plsc.subcore_barrier()
    pltpu.sync_copy(
        acc_sh.at[pl.ds(s * ROWS_PER_TILE, ROWS_PER_TILE)],
        s1_hbm.at[pl.ds(c * ROWS_PAD + s * ROWS_PER_TILE, ROWS_PER_TILE)],
    )


@functools.partial(
    pl.kernel,
    out_type=jax.ShapeDtypeStruct((NC * M_PAD,), jnp.float32),
    mesh=_mesh,
    scratch_types=[
        pltpu.VMEM((IDXBLK, CHUNK), jnp.int32),
        pltpu.VMEM((IDXBLK, CHUNK), jnp.int32),
        [pltpu.VMEM((CHUNK,), jnp.float32)] * 4,
        [pltpu.VMEM((CHUNK,), jnp.int32)] * 4,
        pltpu.VMEM_SHARED((M_PAD,), jnp.float32),
        pltpu.VMEM_SHARED((DEG_PAD,), jnp.float32),
        [pltpu.SemaphoreType.DMA] * 4,
        [pltpu.SemaphoreType.DMA] * 4,
    ],
)
def _m_kernel(src_hbm, dst_hbm, dinv_hbm, z1d_hbm, m_hbm,
              idx_s, idx_d, dval, midx, m_sh, dinv_sh, sv, sm):
    c = lax.axis_index("c")
    s = lax.axis_index("s")
    wid = s * NC + c

    pltpu.sync_copy(
        dinv_hbm.at[pl.ds(s * 640, 640)],
        dinv_sh.at[pl.ds(s * 640, 640)],
    )
    for t in range(M_PER_TILE // M_CHUNK):
        pltpu.sync_copy(
            z1d_hbm.at[pl.ds(s * M_PER_TILE + t * M_CHUNK, M_CHUNK)],
            m_sh.at[pl.ds(s * M_PER_TILE + t * M_CHUNK, M_CHUNK)],
        )
    plsc.subcore_barrier()

    def blk(b, carry):
        pltpu.sync_copy(src_hbm.at[wid, b], idx_s)
        pltpu.sync_copy(dst_hbm.at[wid, b], idx_d)
        for k in range(2):
            pltpu.async_copy(dinv_sh.at[idx_d.at[k]], dval[k], sv[k])
        for k in range(IDXBLK):
            bk = k % 4
            pltpu.make_async_copy(
                dinv_sh.at[idx_d.at[k]], dval[bk], sv[bk]).wait()
            for j in range(CHUNK // 16):
                sl = pl.ds(j * 16, 16)
                d16 = idx_d[k, sl]
                s16 = idx_s[k, sl]
                grp = lax.shift_right_logical(
                    d16 * 10486, jnp.full((16,), 22, jnp.int32))
                midx[bk][sl] = grp * N + s16
            pltpu.async_copy(dval[bk], m_sh.at[midx[bk]], sm[bk], add=True)
            if k + 2 < IDXBLK:
                nb = (k + 2) % 4
                if k >= 2:
                    pltpu.make_async_copy(
                        dval[nb], m_sh.at[midx[nb]], sm[nb]).wait()
                pltpu.async_copy(dinv_sh.at[idx_d.at[k + 2]], dval[nb], sv[nb])
        for k in range(IDXBLK - 4, IDXBLK):
            bk = k % 4
            pltpu.make_async_copy(
                dval[bk], m_sh.at[midx[bk]], sm[bk]).wait()
        return carry

    lax.fori_loop(0, NBLK, blk, 0)
    plsc.subcore_barrier()
    for t in range(M_PER_TILE // M_CHUNK):
        pltpu.sync_copy(
            m_sh.at[pl.ds(s * M_PER_TILE + t * M_CHUNK, M_CHUNK)],
            m_hbm.at[pl.ds(c * M_PAD + s * M_PER_TILE + t * M_CHUNK, M_CHUNK)],
        )


_PRE_BLK = 2000
_PRE_GRID = N // _PRE_BLK


def _tc_pre_body(x_ref, w1_ref, dp0b_ref, dp1b_ref, dp0f_ref, dp1f_ref,
                 g1_ref, hself_ref, dinv_ref, invdeg_ref):
    i = pl.program_id(0)
    h1 = jnp.dot(x_ref[...], w1_ref[...],
                 preferred_element_type=jnp.float32)
    degb = dp0b_ref[...] + dp1b_ref[...] + 1.0
    g1_ref[...] = h1 * (1.0 / jnp.sqrt(degb))
    hself_ref[...] = h1 * (1.0 / degb)

    @pl.when(i == 0)
    def _():
        degf = dp0f_ref[...] + dp1f_ref[...] + 1.0
        dinv_ref[...] = 1.0 / jnp.sqrt(degf)
        invdeg_ref[...] = 1.0 / degf


_tc_pre = pl.pallas_call(
    _tc_pre_body,
    grid=(_PRE_GRID,),
    in_specs=[
        pl.BlockSpec((_PRE_BLK, D), lambda i: (i, 0)),
        pl.BlockSpec((D, D), lambda i: (0, 0)),
        pl.BlockSpec((_PRE_BLK, 1), lambda i: (i, 0)),
        pl.BlockSpec((_PRE_BLK, 1), lambda i: (i, 0)),
        pl.BlockSpec((DEG_PAD, 1), lambda i: (0, 0)),
        pl.BlockSpec((DEG_PAD, 1), lambda i: (0, 0)),
    ],
    out_specs=(
        pl.BlockSpec((_PRE_BLK, D), lambda i: (i, 0)),
        pl.BlockSpec((_PRE_BLK, D), lambda i: (i, 0)),
        pl.BlockSpec((DEG_PAD, 1), lambda i: (0, 0)),
        pl.BlockSpec((DEG_PAD, 1), lambda i: (0, 0)),
    ),
    out_shape=(
        jax.ShapeDtypeStruct((N, D), jnp.float32),
        jax.ShapeDtypeStruct((N, D), jnp.float32),
        jax.ShapeDtypeStruct((DEG_PAD, 1), jnp.float32),
        jax.ShapeDtypeStruct((DEG_PAD, 1), jnp.float32),
    ),
)


def _tc_post_body(s0_ref, s1_ref, hself_ref, dinv_ref, invdeg_ref,
                  m0_ref, m1_ref, w2_ref, b1_ref, b2_ref, out_ref):
    dinv = dinv_ref[...][:N]
    invdeg = invdeg_ref[...][:N]
    s1 = s0_ref[...] + s1_ref[...]
    hr = jnp.maximum(dinv * s1 + hself_ref[...] + b1_ref[...], 0.0)
    t = dinv * hr
    u = invdeg * hr
    m = m0_ref[...] + m1_ref[...]
    agg = (jnp.dot(m, t, preferred_element_type=jnp.float32)
           + jnp.sum(u.reshape(NGRP, GRP, D), axis=1))
    out = jnp.dot(agg, w2_ref[...], preferred_element_type=jnp.float32,
                  precision=_HIGH)
    out_ref[...] = out * (1.0 / GRP) + b2_ref[...]


_tc_post = pl.pallas_call(
    _tc_post_body,
    out_shape=jax.ShapeDtypeStruct((NGRP, 40), jnp.float32),
)


def kernel(x, edge_index, W1, b1, W2, b2):
    src = edge_index[0].reshape(NW, NBLK, IDXBLK, CHUNK)
    dst3 = edge_index[1].reshape(NW, NBLK, IDXBLK, CHUNK)

    degp = _deg_kernel(dst3)
    dp = degp.reshape(NC, DEG_PAD, 1)
    g1, hself, dinv2d, invdeg2d = _tc_pre(x, W1, dp[0], dp[1], dp[0], dp[1])

    z2d = jnp.zeros((ROWS_PAD, D), jnp.float32)
    z1d = jnp.zeros((M_PAD,), jnp.float32)
    s1p = _edge_kernel(src, dst3, g1, z2d)
    mp = _m_kernel(src, dst3, dinv2d.reshape(DEG_PAD), z1d)

    s1p = s1p.reshape(NC, ROWS_PAD, D)
    mp = mp.reshape(NC, M_PAD)
    m0 = mp[0, : NGRP * N].reshape(NGRP, N)
    m1 = mp[1, : NGRP * N].reshape(NGRP, N)

    out = _tc_post(s1p[0, :N], s1p[1, :N], hself, dinv2d, invdeg2d,
                   m0, m1, W2, b1.reshape(1, D), b2.reshape(1, 40))
    return out
